# baseline (speedup 1.0000x reference)
import functools

import jax
import jax.numpy as jnp
from jax import lax
from jax.experimental import pallas as pl
from jax.experimental.pallas import tpu as pltpu
from jax.experimental.pallas import tpu_sc as plsc

N = 10000
D = 128
H = 128

NC = 2
NS = 16
NW = NC * NS

CH = 128
UP_CPW = 80
B_CPW = 8

NPAD = 10240
ROWS_PER_TILE = NPAD // NS
ZROWS = 16
IDX_STAGE = 40


def _sc_segment_sum(table, src, dst, cpw):
    mesh = plsc.VectorSubcoreMesh(
        core_axis_name="c", subcore_axis_name="s",
        num_cores=NC, num_subcores=NS)

    @functools.partial(
        pl.kernel,
        out_type=jax.ShapeDtypeStruct((NC, NPAD, D), jnp.float32),
        mesh=mesh,
        scratch_types=[
            pltpu.VMEM_SHARED((NPAD, D), jnp.float32),
            pltpu.VMEM((IDX_STAGE, CH), jnp.int32),
            pltpu.VMEM((IDX_STAGE, CH), jnp.int32),
            pltpu.VMEM((CH, D), jnp.float32),
            pltpu.VMEM((CH, D), jnp.float32),
            pltpu.VMEM((ZROWS, D), jnp.float32),
            pltpu.SemaphoreType.DMA,
            pltpu.SemaphoreType.DMA,
            pltpu.SemaphoreType.DMA,
        ],
    )
    def k(table_hbm, src_hbm, dst_hbm,
          out_hbm, acc, src_idx, dst_idx, rows0, rows1, zbuf, sem0, sem1,
          zsem):
        c = lax.axis_index("c")
        s = lax.axis_index("s")
        w = c * NS + s
        row0 = s * ROWS_PER_TILE

        def zrow(r, carry):
            for cc in range(D // 16):
                zbuf[r, pl.ds(cc * 16, 16)] = jnp.zeros((16,), jnp.float32)
            return carry
        lax.fori_loop(0, ZROWS, zrow, 0)

        nz = ROWS_PER_TILE // ZROWS

        def zissue(kk, carry):
            pltpu.async_copy(
                zbuf, acc.at[pl.ds(row0 + kk * ZROWS, ZROWS)], zsem)
            return carry
        lax.fori_loop(0, nz, zissue, 0)

        def zdrain(kk, carry):
            pltpu.make_async_copy(
                zbuf, acc.at[pl.ds(row0 + kk * ZROWS, ZROWS)], zsem).wait()
            return carry
        lax.fori_loop(0, nz, zdrain, 0)

        plsc.subcore_barrier()

        bufs = (rows0, rows1)
        sems = (sem0, sem1)
        nstages = (cpw + IDX_STAGE - 1) // IDX_STAGE
        for st in range(nstages):
            sc = min(IDX_STAGE, cpw - st * IDX_STAGE)
            base = w * cpw + st * IDX_STAGE
            pltpu.sync_copy(src_hbm.at[pl.ds(base, sc)],
                            src_idx.at[pl.ds(0, sc)])
            pltpu.sync_copy(dst_hbm.at[pl.ds(base, sc)],
                            dst_idx.at[pl.ds(0, sc)])
            pltpu.async_copy(table_hbm.at[src_idx.at[0]], rows0, sem0)
            pltpu.async_copy(table_hbm.at[src_idx.at[1]], rows1, sem1)
            npairs = sc // 2

            def body(t, carry):
                for b in range(2):
                    j = 2 * t + b
                    pltpu.make_async_copy(
                        table_hbm.at[src_idx.at[j]], bufs[b],
                        sems[b]).wait()
                    pltpu.sync_copy(bufs[b], acc.at[dst_idx.at[j]],
                                    add=True)

                    @pl.when(t < npairs - 1)
                    def _():
                        pltpu.async_copy(
                            table_hbm.at[src_idx.at[j + 2]], bufs[b],
                            sems[b])
                return carry
            lax.fori_loop(0, npairs, body, 0)

        plsc.subcore_barrier()
        pltpu.sync_copy(acc.at[pl.ds(row0, ROWS_PER_TILE)],
                        out_hbm.at[c, pl.ds(row0, ROWS_PER_TILE)])

    return k(table, src, dst)


def _bn_relu(h, gamma, beta):
    m = jnp.mean(h, axis=0, keepdims=True)
    v = jnp.mean((h - m) ** 2, axis=0, keepdims=True)
    return jnp.maximum(gamma * (h - m) / jnp.sqrt(v + 1e-5) + beta, 0.0)


def _mlp_body(x_ref, parts_ref, W1_ref, b1_ref, g1_ref, be1_ref,
              W2_ref, b2_ref, g2_ref, be2_ref, eps_ref, o_ref):
    agg = parts_ref[0, :N] + parts_ref[1, :N]
    h = agg + (1.0 + eps_ref[0, 0]) * x_ref[...]
    h = _bn_relu(jnp.dot(h, W1_ref[...], preferred_element_type=jnp.float32)
                 + b1_ref[...], g1_ref[...], be1_ref[...])
    h = _bn_relu(jnp.dot(h, W2_ref[...], preferred_element_type=jnp.float32)
                 + b2_ref[...], g2_ref[...], be2_ref[...])
    o_ref[...] = h


def _final_body(u_ref, b_ref, Wcu_ref, Wcb_ref, bc_ref, gc_ref, bec_ref,
                o_ref):
    catw = (jnp.dot(u_ref[...], Wcu_ref[...],
                    preferred_element_type=jnp.float32)
            + jnp.dot(b_ref[...], Wcb_ref[...],
                      preferred_element_type=jnp.float32)
            + bc_ref[...])
    o_ref[...] = _bn_relu(catw, gc_ref[...], bec_ref[...])


_TC_PARAMS = pltpu.CompilerParams(vmem_limit_bytes=100 * 1024 * 1024)


def _mlp(x, parts, W1, b1, g1, be1, W2, b2, g2, be2, eps1):
    row = lambda a: a.reshape(1, -1)
    return pl.pallas_call(
        _mlp_body,
        out_shape=jax.ShapeDtypeStruct((N, H), jnp.float32),
        compiler_params=_TC_PARAMS,
    )(x, parts, W1, row(b1), row(g1), row(be1),
      W2, row(b2), row(g2), row(be2), eps1.reshape(1, 1))


def _pad_idx(idx, total, srcs=False):
    pad = total - idx.shape[0]
    ar = jnp.arange(pad, dtype=jnp.int32)
    if srcs:
        tail = ar % N
    else:
        tail = N + ar % (NPAD - N)
    idx = jnp.concatenate([idx, tail])
    return idx.reshape(-1, CH)


def kernel(x, up_index, up_attr, boundary_attr, boundary_index,
           W1u, b1u, g1u, be1u, W2u, b2u, g2u, be2u,
           W1b, b1b, g1b, be1b, W2b, b2b, g2b, be2b,
           Wc, bc, gc, bec, eps1):
    up_src = _pad_idx(up_index[0], UP_CPW * NW * CH, srcs=True)
    up_dst = _pad_idx(up_index[1], UP_CPW * NW * CH)
    b_src = _pad_idx(boundary_index[0], B_CPW * NW * CH, srcs=True)
    b_dst = _pad_idx(boundary_index[1], B_CPW * NW * CH)

    parts_up = _sc_segment_sum(x, up_src, up_dst, UP_CPW)
    parts_b = _sc_segment_sum(boundary_attr, b_src, b_dst, B_CPW)

    out_up = _mlp(x, parts_up, W1u, b1u, g1u, be1u, W2u, b2u, g2u, be2u,
                  eps1)
    out_b = _mlp(x, parts_b, W1b, b1b, g1b, be1b, W2b, b2b, g2b, be2b,
                 eps1)

    row = lambda a: a.reshape(1, -1)
    return pl.pallas_call(
        _final_body,
        out_shape=jax.ShapeDtypeStruct((N, H), jnp.float32),
        compiler_params=_TC_PARAMS,
    )(out_up, out_b, Wc[:H], Wc[H:], row(bc), row(gc), row(bec))


# speedup vs baseline: 1.0262x; 1.0262x over previous
import functools

import jax
import jax.numpy as jnp
from jax import lax
from jax.experimental import pallas as pl
from jax.experimental.pallas import tpu as pltpu
from jax.experimental.pallas import tpu_sc as plsc

N = 10000
D = 128
H = 128

NC = 2
NS = 16
NW = NC * NS

CH = 128
UP_CPW = 80
B_CPW = 8

NPAD = 10240
ROWS_PER_TILE = NPAD // NS
ZROWS = 16
IDX_STAGE = 40


def _sc_segment_sums(xpad, battr_pad, up_src, up_dst, b_src, b_dst):
    mesh = plsc.VectorSubcoreMesh(
        core_axis_name="c", subcore_axis_name="s",
        num_cores=NC, num_subcores=NS)

    @functools.partial(
        pl.kernel,
        out_type=jax.ShapeDtypeStruct((2 * NC, NPAD, D), jnp.float32),
        mesh=mesh,
        scratch_types=[
            pltpu.VMEM_SHARED((NPAD, D), jnp.float32),
            pltpu.VMEM((IDX_STAGE, CH), jnp.int32),
            pltpu.VMEM((IDX_STAGE, CH), jnp.int32),
            pltpu.VMEM((CH, D), jnp.float32),
            pltpu.VMEM((CH, D), jnp.float32),
            pltpu.VMEM((ZROWS, D), jnp.float32),
            pltpu.SemaphoreType.DMA,
            pltpu.SemaphoreType.DMA,
            pltpu.SemaphoreType.DMA,
        ],
    )
    def k(xpad_hbm, battr_hbm, up_src_hbm, up_dst_hbm, b_src_hbm, b_dst_hbm,
          out_hbm, acc, src_idx, dst_idx, rows0, rows1, zbuf, sem0, sem1,
          zsem):
        c = lax.axis_index("c")
        s = lax.axis_index("s")
        w = c * NS + s
        row0 = s * ROWS_PER_TILE

        def zrow(r, carry):
            for cc in range(D // 16):
                zbuf[r, pl.ds(cc * 16, 16)] = jnp.zeros((16,), jnp.float32)
            return carry
        lax.fori_loop(0, ZROWS, zrow, 0)

        def zero_acc():
            nz = ROWS_PER_TILE // ZROWS

            def zissue(kk, carry):
                pltpu.async_copy(
                    zbuf, acc.at[pl.ds(row0 + kk * ZROWS, ZROWS)], zsem)
                return carry
            lax.fori_loop(0, nz, zissue, 0)

            def zdrain(kk, carry):
                pltpu.make_async_copy(
                    zbuf, acc.at[pl.ds(row0 + kk * ZROWS, ZROWS)], zsem).wait()
                return carry
            lax.fori_loop(0, nz, zdrain, 0)

        bufs = (rows0, rows1)
        sems = (sem0, sem1)

        def run_phase(table_hbm, src_hbm, dst_hbm, cpw):
            nstages = (cpw + IDX_STAGE - 1) // IDX_STAGE
            for st in range(nstages):
                sc = min(IDX_STAGE, cpw - st * IDX_STAGE)
                base = w * cpw + st * IDX_STAGE
                pltpu.sync_copy(src_hbm.at[pl.ds(base, sc)],
                                src_idx.at[pl.ds(0, sc)])
                pltpu.sync_copy(dst_hbm.at[pl.ds(base, sc)],
                                dst_idx.at[pl.ds(0, sc)])
                pltpu.async_copy(table_hbm.at[src_idx.at[0]], rows0, sem0)
                pltpu.async_copy(table_hbm.at[src_idx.at[1]], rows1, sem1)
                npairs = sc // 2

                def body(t, carry):
                    for b in range(2):
                        j = 2 * t + b
                        pltpu.make_async_copy(
                            table_hbm.at[src_idx.at[j]], bufs[b],
                            sems[b]).wait()
                        pltpu.sync_copy(bufs[b], acc.at[dst_idx.at[j]],
                                        add=True)

                        @pl.when(t < npairs - 1)
                        def _():
                            pltpu.async_copy(
                                table_hbm.at[src_idx.at[j + 2]], bufs[b],
                                sems[b])
                    return carry
                lax.fori_loop(0, npairs, body, 0)

        def writeback(slot):
            pltpu.sync_copy(acc.at[pl.ds(row0, ROWS_PER_TILE)],
                            out_hbm.at[slot, pl.ds(row0, ROWS_PER_TILE)])

        zero_acc()
        plsc.subcore_barrier()
        run_phase(xpad_hbm, up_src_hbm, up_dst_hbm, UP_CPW)
        plsc.subcore_barrier()
        writeback(c)
        zero_acc()
        plsc.subcore_barrier()
        run_phase(battr_hbm, b_src_hbm, b_dst_hbm, B_CPW)
        plsc.subcore_barrier()
        writeback(NC + c)

    return k(xpad, battr_pad, up_src, up_dst, b_src, b_dst)


def _bn_relu(h, gamma, beta):
    m = jnp.mean(h, axis=0, keepdims=True)
    v = jnp.mean((h - m) ** 2, axis=0, keepdims=True)
    return jnp.maximum(gamma * (h - m) / jnp.sqrt(v + 1e-5) + beta, 0.0)


def _dense_body(x_ref, parts_ref,
                W1u_ref, b1u_ref, g1u_ref, be1u_ref,
                W2u_ref, b2u_ref, g2u_ref, be2u_ref,
                W1b_ref, b1b_ref, g1b_ref, be1b_ref,
                W2b_ref, b2b_ref, g2b_ref, be2b_ref,
                Wcu_ref, Wcb_ref, bc_ref, gc_ref, bec_ref, eps_ref, o_ref):
    xv = x_ref[...]
    scale = 1.0 + eps_ref[0, 0]
    agg_up = parts_ref[0, :N] + parts_ref[1, :N]
    agg_b = parts_ref[2, :N] + parts_ref[3, :N]

    def mlp(h, W1, b1, g1, be1, W2, b2, g2, be2):
        h = _bn_relu(jnp.dot(h, W1, preferred_element_type=jnp.float32) + b1,
                     g1, be1)
        h = _bn_relu(jnp.dot(h, W2, preferred_element_type=jnp.float32) + b2,
                     g2, be2)
        return h

    out_up = mlp(agg_up + scale * xv,
                 W1u_ref[...], b1u_ref[...], g1u_ref[...], be1u_ref[...],
                 W2u_ref[...], b2u_ref[...], g2u_ref[...], be2u_ref[...])
    out_b = mlp(agg_b + scale * xv,
                W1b_ref[...], b1b_ref[...], g1b_ref[...], be1b_ref[...],
                W2b_ref[...], b2b_ref[...], g2b_ref[...], be2b_ref[...])
    catw = (jnp.dot(out_up, Wcu_ref[...], preferred_element_type=jnp.float32)
            + jnp.dot(out_b, Wcb_ref[...], preferred_element_type=jnp.float32)
            + bc_ref[...])
    o_ref[...] = _bn_relu(catw, gc_ref[...], bec_ref[...])


def _pad_idx(idx, total, srcs=False):
    pad = total - idx.shape[0]
    ar = jnp.arange(pad, dtype=jnp.int32)
    if srcs:
        tail = ar % N
    else:
        tail = N + ar % (NPAD - N)
    idx = jnp.concatenate([idx, tail])
    return idx.reshape(-1, CH)


def kernel(x, up_index, up_attr, boundary_attr, boundary_index,
           W1u, b1u, g1u, be1u, W2u, b2u, g2u, be2u,
           W1b, b1b, g1b, be1b, W2b, b2b, g2b, be2b,
           Wc, bc, gc, bec, eps1):
    up_src = _pad_idx(up_index[0], UP_CPW * NW * CH, srcs=True)
    up_dst = _pad_idx(up_index[1], UP_CPW * NW * CH)
    b_src = _pad_idx(boundary_index[0], B_CPW * NW * CH, srcs=True)
    b_dst = _pad_idx(boundary_index[1], B_CPW * NW * CH)

    parts = _sc_segment_sums(x, boundary_attr, up_src, up_dst, b_src, b_dst)

    row = lambda a: a.reshape(1, -1)
    return pl.pallas_call(
        _dense_body,
        out_shape=jax.ShapeDtypeStruct((N, H), jnp.float32),
        compiler_params=pltpu.CompilerParams(
            vmem_limit_bytes=120 * 1024 * 1024),
    )(x, parts,
      W1u, row(b1u), row(g1u), row(be1u),
      W2u, row(b2u), row(g2u), row(be2u),
      W1b, row(b1b), row(g1b), row(be1b),
      W2b, row(b2b), row(g2b), row(be2b),
      Wc[:H], Wc[H:], row(bc), row(gc), row(bec),
      eps1.reshape(1, 1))
